# baseline (speedup 1.0000x reference)
import jax
import jax.numpy as jnp
from jax import lax
from jax.experimental import pallas as pl
from jax.experimental.pallas import tpu as pltpu
from jax.experimental.pallas import tpu_sc as plsc

DIM = 1024
SCALE = DIM ** (-0.5)

_NC = 2
_NS = 16
_NW = _NC * _NS

_SC_ROWS = 2048
_PER_W = _SC_ROWS // _NW
_CHUNK = 16
_NBUF = 4
_NCHUNK = _PER_W // _CHUNK
_LAG = 1

_TC_BLOCK = 2048


def _sc_body(w_hbm, out_hbm, buf, in_sem, out_sem):
    wid = lax.axis_index("s") * _NC + lax.axis_index("c")
    row0 = wid * _PER_W

    def start_in(i):
        b = i % _NBUF
        return pltpu.async_copy(
            w_hbm.at[pl.ds(row0 + i * _CHUNK, _CHUNK)],
            buf.at[b],
            in_sem.at[b],
        )

    def start_out(i):
        b = i % _NBUF
        return pltpu.async_copy(
            buf.at[b],
            out_hbm.at[pl.ds(row0 + i * _CHUNK, _CHUNK)],
            out_sem.at[b],
        )

    in_descs = {}
    out_descs = {}
    unwaited_out = set()

    for i in range(min(_NBUF, _NCHUNK)):
        in_descs[i] = start_in(i)

    for i in range(_NCHUNK):
        b = i % _NBUF
        j = i - _LAG
        if j >= 0 and j + _NBUF < _NCHUNK:
            out_descs[j].wait()
            unwaited_out.discard(j)
            in_descs[j + _NBUF] = start_in(j + _NBUF)

        in_descs[i].wait()

        @plsc.parallel_loop(0, _CHUNK, 1)
        def _(r):
            for c in range(0, DIM, 16):
                buf[b, r, pl.ds(c, 16)] = buf[b, r, pl.ds(c, 16)] * SCALE

        out_descs[i] = start_out(i)
        unwaited_out.add(i)

    for i in sorted(unwaited_out):
        out_descs[i].wait()


def _tc_body(w_ref, o_ref):
    o_ref[...] = w_ref[...] * SCALE


def kernel(x, W):
    n = x.shape[1]
    mesh = plsc.VectorSubcoreMesh(core_axis_name="c", subcore_axis_name="s")
    sc_out = pl.kernel(
        _sc_body,
        out_type=jax.ShapeDtypeStruct((_SC_ROWS, DIM), jnp.float32),
        mesh=mesh,
        scratch_types=[
            pltpu.VMEM((_NBUF, _CHUNK, DIM), jnp.float32),
            pltpu.SemaphoreType.DMA((_NBUF,)),
            pltpu.SemaphoreType.DMA((_NBUF,)),
        ],
    )(W[:_SC_ROWS])

    tc_rows = n - _SC_ROWS
    tc_out = pl.pallas_call(
        _tc_body,
        grid=(tc_rows // _TC_BLOCK,),
        in_specs=[pl.BlockSpec((_TC_BLOCK, DIM), lambda i: (i, 0))],
        out_specs=pl.BlockSpec((_TC_BLOCK, DIM), lambda i: (i, 0)),
        out_shape=jax.ShapeDtypeStruct((tc_rows, DIM), W.dtype),
    )(W[_SC_ROWS:n])

    return jnp.concatenate([sc_out, tc_out], axis=0)


# speedup vs baseline: 3.3163x; 3.3163x over previous
import jax
import jax.numpy as jnp
from jax.experimental import pallas as pl
from jax.experimental.pallas import tpu as pltpu

DIM = 1024
SCALE = DIM ** (-0.5)

_ROWS = 8192
_CHUNK = 512
_NBUF = 4
_NCHUNK = _ROWS // _CHUNK
_LAG = 1


def _tc_body(w_hbm, o_hbm, buf, in_sem, out_sem):
    def start_in(i):
        b = i % _NBUF
        return pltpu.make_async_copy(
            w_hbm.at[pl.ds(i * _CHUNK, _CHUNK)],
            buf.at[b],
            in_sem.at[b],
        )

    def start_out(i):
        b = i % _NBUF
        return pltpu.make_async_copy(
            buf.at[b],
            o_hbm.at[pl.ds(i * _CHUNK, _CHUNK)],
            out_sem.at[b],
        )

    unwaited = set()
    for i in range(_NBUF):
        start_in(i).start()

    for i in range(_NCHUNK):
        b = i % _NBUF
        j = i - _LAG
        if j >= 0 and j + _NBUF < _NCHUNK:
            start_out(j).wait()
            unwaited.discard(j)
            start_in(j + _NBUF).start()

        start_in(i).wait()
        buf[b] = buf[b] * SCALE
        start_out(i).start()
        unwaited.add(i)

    for i in sorted(unwaited):
        start_out(i).wait()


def kernel(x, W):
    n = x.shape[1]
    return pl.pallas_call(
        _tc_body,
        in_specs=[pl.BlockSpec(memory_space=pl.ANY)],
        out_specs=pl.BlockSpec(memory_space=pl.ANY),
        out_shape=jax.ShapeDtypeStruct((n, DIM), W.dtype),
        scratch_shapes=[
            pltpu.VMEM((_NBUF, _CHUNK, DIM), jnp.float32),
            pltpu.SemaphoreType.DMA((_NBUF,)),
            pltpu.SemaphoreType.DMA((_NBUF,)),
        ],
    )(W[:n])


# speedup vs baseline: 3.9919x; 1.2037x over previous
import jax
import jax.numpy as jnp
from jax.experimental import pallas as pl
from jax.experimental.pallas import tpu as pltpu

DIM = 1024
SCALE = DIM ** (-0.5)

_ROWS = 8192
_CHUNK = 512
_NBUF = 6
_NCHUNK = _ROWS // _CHUNK
_LAG = 3


def _tc_body(w_hbm, o_hbm, buf, in_sem, out_sem):
    def start_in(i):
        b = i % _NBUF
        return pltpu.make_async_copy(
            w_hbm.at[pl.ds(i * _CHUNK, _CHUNK)],
            buf.at[b],
            in_sem.at[b],
        )

    def start_out(i):
        b = i % _NBUF
        return pltpu.make_async_copy(
            buf.at[b],
            o_hbm.at[pl.ds(i * _CHUNK, _CHUNK)],
            out_sem.at[b],
        )

    unwaited = set()
    for i in range(_NBUF):
        start_in(i).start()

    for i in range(_NCHUNK):
        b = i % _NBUF
        j = i - _LAG
        if j >= 0 and j + _NBUF < _NCHUNK:
            start_out(j).wait()
            unwaited.discard(j)
            start_in(j + _NBUF).start()

        start_in(i).wait()
        buf[b] = buf[b] * SCALE
        start_out(i).start()
        unwaited.add(i)

    for i in sorted(unwaited):
        start_out(i).wait()


def kernel(x, W):
    n = x.shape[1]
    return pl.pallas_call(
        _tc_body,
        in_specs=[pl.BlockSpec(memory_space=pl.ANY)],
        out_specs=pl.BlockSpec(memory_space=pl.ANY),
        out_shape=jax.ShapeDtypeStruct((n, DIM), W.dtype),
        scratch_shapes=[
            pltpu.VMEM((_NBUF, _CHUNK, DIM), jnp.float32),
            pltpu.SemaphoreType.DMA((_NBUF,)),
            pltpu.SemaphoreType.DMA((_NBUF,)),
        ],
    )(W[:n])


# speedup vs baseline: 4.0258x; 1.0085x over previous
import jax
import jax.numpy as jnp
from jax.experimental import pallas as pl
from jax.experimental.pallas import tpu as pltpu

DIM = 1024
SCALE = DIM ** (-0.5)

_ROWS = 8192
_CHUNK = 1024
_NBUF = 6
_NCHUNK = _ROWS // _CHUNK
_LAG = 3


def _tc_body(w_hbm, o_hbm, buf, in_sem, out_sem):
    def start_in(i):
        b = i % _NBUF
        return pltpu.make_async_copy(
            w_hbm.at[pl.ds(i * _CHUNK, _CHUNK)],
            buf.at[b],
            in_sem.at[b],
        )

    def start_out(i):
        b = i % _NBUF
        return pltpu.make_async_copy(
            buf.at[b],
            o_hbm.at[pl.ds(i * _CHUNK, _CHUNK)],
            out_sem.at[b],
        )

    unwaited = set()
    for i in range(_NBUF):
        start_in(i).start()

    for i in range(_NCHUNK):
        b = i % _NBUF
        j = i - _LAG
        if j >= 0 and j + _NBUF < _NCHUNK:
            start_out(j).wait()
            unwaited.discard(j)
            start_in(j + _NBUF).start()

        start_in(i).wait()
        buf[b] = buf[b] * SCALE
        start_out(i).start()
        unwaited.add(i)

    for i in sorted(unwaited):
        start_out(i).wait()


def kernel(x, W):
    n = x.shape[1]
    return pl.pallas_call(
        _tc_body,
        in_specs=[pl.BlockSpec(memory_space=pl.ANY)],
        out_specs=pl.BlockSpec(memory_space=pl.ANY),
        out_shape=jax.ShapeDtypeStruct((n, DIM), W.dtype),
        scratch_shapes=[
            pltpu.VMEM((_NBUF, _CHUNK, DIM), jnp.float32),
            pltpu.SemaphoreType.DMA((_NBUF,)),
            pltpu.SemaphoreType.DMA((_NBUF,)),
        ],
    )(W[:n])


# speedup vs baseline: 4.0942x; 1.0170x over previous
import jax
import jax.numpy as jnp
from jax.experimental import pallas as pl

DIM = 1024
SCALE = DIM ** (-0.5)


def _scale_kernel(w_ref, o_ref):
    o_ref[...] = w_ref[...] * SCALE


def kernel(x, W):
    n = x.shape[1]
    rows_per_block = 2048
    grid = (n // rows_per_block,)
    return pl.pallas_call(
        _scale_kernel,
        grid=grid,
        in_specs=[pl.BlockSpec((rows_per_block, DIM), lambda i: (i, 0))],
        out_specs=pl.BlockSpec((rows_per_block, DIM), lambda i: (i, 0)),
        out_shape=jax.ShapeDtypeStruct((n, DIM), W.dtype),
    )(W[:n])
